# baseline (speedup 1.0000x reference)
import functools

import jax
import jax.numpy as jnp
from jax import lax
from jax.experimental import pallas as pl
from jax.experimental.pallas import tpu as pltpu
from jax.experimental.pallas import tpu_sc as plsc

_NC, _NS = 2, 16
_NW = _NC * _NS
_CH = 400
_SUB = 80


def _sc_gather(table, idx_flat):
    e_total = idx_flat.shape[0]
    f = table.shape[1]
    dt = table.dtype
    rows_per_w = e_total // _NW
    n_ch = rows_per_w // _CH
    n_sub = _CH // _SUB
    mesh = plsc.VectorSubcoreMesh(core_axis_name="c", subcore_axis_name="s",
                                  num_cores=_NC, num_subcores=_NS)

    @functools.partial(
        pl.kernel, mesh=mesh,
        out_type=jax.ShapeDtypeStruct((e_total, f), dt),
        scratch_types=[
            pltpu.VMEM((_CH,), jnp.int32),
            pltpu.VMEM((_CH,), jnp.int32),
            pltpu.VMEM((_CH, f), dt),
            pltpu.VMEM((_CH, f), dt),
            pltpu.SemaphoreType.DMA,
            pltpu.SemaphoreType.DMA,
            pltpu.SemaphoreType.DMA,
        ],
    )
    def k(table_hbm, idx_hbm, out_hbm, idx_v0, idx_v1, rows_v0, rows_v1,
          isem, gsem, osem):
        wid = lax.axis_index("s") * _NC + lax.axis_index("c")
        base = wid * rows_per_w
        idx_vs = (idx_v0, idx_v1)
        rows_vs = (rows_v0, rows_v1)

        def idx_cp(i, slot):
            return pltpu.make_async_copy(
                idx_hbm.at[pl.ds(base + i * _CH, _CH)], idx_vs[slot], isem)

        def gather_cps(slot):
            return [pltpu.make_async_copy(
                table_hbm.at[idx_vs[slot].at[pl.ds(j * _SUB, _SUB)]],
                rows_vs[slot].at[pl.ds(j * _SUB, _SUB)], gsem)
                for j in range(n_sub)]

        def out_cp(i, slot):
            return pltpu.make_async_copy(
                rows_vs[slot], out_hbm.at[pl.ds(base + i * _CH, _CH)], osem)

        def body(i, carry):
            idx_cp(i, 0).start()
            idx_cp(i, 0).wait()
            copies = gather_cps(0)
            for cp in copies:
                cp.start()
            for cp in copies:
                cp.wait()
            out_cp(i, 0).start()
            out_cp(i, 0).wait()
            return carry

        lax.fori_loop(0, n_ch, body, 0)

    return k(table, idx_flat)


def _edge_preact(ag, a_blk, b3, ws, wn, wb, b, nb, m):
    two_f = ws.shape[1]
    de = b3.shape[2]
    bf = jnp.bfloat16
    s = jnp.dot(a_blk.astype(bf), ws, preferred_element_type=jnp.float32)
    ynb = jnp.dot(ag.astype(bf), wn, preferred_element_type=jnp.float32)
    q = jnp.dot(b3.reshape(nb * m, de), wb, preferred_element_type=jnp.float32)
    s_exp = jnp.broadcast_to(s[:, None, :], (nb, m, two_f)).reshape(nb * m, two_f)
    return ynb + q + s_exp + b


def _stats_body(nb, m, ag_ref, a_ref, b3_ref, ws_ref, wn_ref, wb_ref,
                b_ref, out_ref):
    y = _edge_preact(ag_ref[...], a_ref[...], b3_ref[...],
                     ws_ref[...], wn_ref[...], wb_ref[...], b_ref[...], nb, m)

    @pl.when(pl.program_id(0) == 0)
    def _():
        out_ref[...] = jnp.zeros_like(out_ref)

    ssum = jnp.sum(y, axis=0, keepdims=True)
    ssq = jnp.sum(y * y, axis=0, keepdims=True)
    out_ref[...] += jnp.concatenate([ssum, ssq], axis=0)


def _main_body(nb, m, e_total, stats_ref, ag_ref, a_ref, b3_ref, ws_ref,
               wn_ref, wb_ref, b_ref, g1_ref, b1_ref, out_ref):
    y = _edge_preact(ag_ref[...], a_ref[...], b3_ref[...],
                     ws_ref[...], wn_ref[...], wb_ref[...], b_ref[...], nb, m)
    stats = stats_ref[...]
    mean = stats[0:1, :] / e_total
    var = stats[1:2, :] / e_total - mean * mean
    inv = lax.rsqrt(var + 1e-5)
    z = (y - mean) * (inv * g1_ref[...]) + b1_ref[...]
    f = z.shape[1] // 2
    filt = jax.nn.sigmoid(z[:, :f])
    core = jax.nn.softplus(z[:, f:])
    p = (filt * core).reshape(nb, m, f)
    out_ref[...] = jnp.sum(p, axis=1)


def _final_body(n, a_ref, ps_ref, g2_ref, b2_ref, out_ref):
    x = ps_ref[...]
    mean = jnp.mean(x, axis=0, keepdims=True)
    d = x - mean
    var = jnp.mean(d * d, axis=0, keepdims=True)
    z = d * lax.rsqrt(var + 1e-5) * g2_ref[...] + b2_ref[...]
    out_ref[...] = jax.nn.softplus(a_ref[...] + z)


def _tc_pipeline(ag, a, b3, ws, wn, wb, b, g1, b1, g2, b2,
                 interpret=False):
    n, f = a.shape
    e_total = ag.shape[0]
    m = e_total // n
    two_f = 2 * f
    de = b3.shape[2]
    nb = 200
    r = nb * m
    grid = n // nb

    full = lambda shape: pl.BlockSpec(shape, lambda i: (0,) * len(shape))
    stats = pl.pallas_call(
        functools.partial(_stats_body, nb, m),
        grid=(grid,),
        in_specs=[
            pl.BlockSpec((r, f), lambda i: (i, 0)),
            pl.BlockSpec((nb, f), lambda i: (i, 0)),
            pl.BlockSpec((nb, m, de), lambda i: (i, 0, 0)),
            full((f, two_f)), full((f, two_f)), full((de, two_f)),
            full((1, two_f)),
        ],
        out_specs=pl.BlockSpec((2, two_f), lambda i: (0, 0)),
        out_shape=jax.ShapeDtypeStruct((2, two_f), jnp.float32),
        interpret=interpret,
    )(ag, a, b3, ws, wn, wb, b)

    presum = pl.pallas_call(
        functools.partial(_main_body, nb, m, float(e_total)),
        grid=(grid,),
        in_specs=[
            full((2, two_f)),
            pl.BlockSpec((r, f), lambda i: (i, 0)),
            pl.BlockSpec((nb, f), lambda i: (i, 0)),
            pl.BlockSpec((nb, m, de), lambda i: (i, 0, 0)),
            full((f, two_f)), full((f, two_f)), full((de, two_f)),
            full((1, two_f)), full((1, two_f)), full((1, two_f)),
        ],
        out_specs=pl.BlockSpec((nb, f), lambda i: (i, 0)),
        out_shape=jax.ShapeDtypeStruct((n, f), jnp.float32),
        interpret=interpret,
    )(stats, ag, a, b3, ws, wn, wb, b, g1, b1)

    out = pl.pallas_call(
        functools.partial(_final_body, n),
        out_shape=jax.ShapeDtypeStruct((n, f), jnp.float32),
        interpret=interpret,
    )(a, presum, g2, b2)
    return out


def kernel(atom_features, bond_features, W, b, bn1_scale, bn1_bias,
           bn2_scale, bn2_bias, neighbor_indices):
    a = atom_features
    n, f = a.shape
    m = neighbor_indices.shape[1]
    idx_flat = neighbor_indices.reshape(-1)
    w16 = W.astype(jnp.bfloat16)
    ws, wn = w16[:f], w16[f:2 * f]
    wb = W[2 * f:]

    ag = _sc_gather(a, idx_flat)
    return _tc_pipeline(
        ag, a, bond_features, ws, wn, wb,
        b.reshape(1, -1), bn1_scale.reshape(1, -1), bn1_bias.reshape(1, -1),
        bn2_scale.reshape(1, -1), bn2_bias.reshape(1, -1))


# speedup vs baseline: 1.0224x; 1.0224x over previous
import functools

import jax
import jax.numpy as jnp
from jax import lax
from jax.experimental import pallas as pl
from jax.experimental.pallas import tpu as pltpu
from jax.experimental.pallas import tpu_sc as plsc

_NC, _NS = 2, 16
_NW = _NC * _NS
_CH = 400
_SUB = 400


def _sc_gather(table, idx_flat):
    e_total = idx_flat.shape[0]
    f = table.shape[1]
    dt = table.dtype
    rows_per_w = e_total // _NW
    n_ch = rows_per_w // _CH
    n_sub = _CH // _SUB
    mesh = plsc.VectorSubcoreMesh(core_axis_name="c", subcore_axis_name="s",
                                  num_cores=_NC, num_subcores=_NS)

    @functools.partial(
        pl.kernel, mesh=mesh,
        out_type=jax.ShapeDtypeStruct((e_total, f), dt),
        scratch_types=[
            pltpu.VMEM((_CH,), jnp.int32),
            pltpu.VMEM((_CH,), jnp.int32),
            pltpu.VMEM((_CH, f), dt),
            pltpu.VMEM((_CH, f), dt),
            pltpu.SemaphoreType.DMA,
            pltpu.SemaphoreType.DMA,
            pltpu.SemaphoreType.DMA,
        ],
    )
    def k(table_hbm, idx_hbm, out_hbm, idx_v0, idx_v1, rows_v0, rows_v1,
          isem, gsem, osem):
        wid = lax.axis_index("s") * _NC + lax.axis_index("c")
        base = wid * rows_per_w
        idx_vs = (idx_v0, idx_v1)
        rows_vs = (rows_v0, rows_v1)

        def idx_cp(i, slot):
            return pltpu.make_async_copy(
                idx_hbm.at[pl.ds(base + i * _CH, _CH)], idx_vs[slot], isem)

        def gather_cps(slot):
            return [pltpu.make_async_copy(
                table_hbm.at[idx_vs[slot].at[pl.ds(j * _SUB, _SUB)]],
                rows_vs[slot].at[pl.ds(j * _SUB, _SUB)], gsem)
                for j in range(n_sub)]

        def out_cp(i, slot):
            return pltpu.make_async_copy(
                rows_vs[slot], out_hbm.at[pl.ds(base + i * _CH, _CH)], osem)

        def body(i, carry):
            idx_cp(i, 0).start()
            idx_cp(i, 0).wait()
            copies = gather_cps(0)
            for cp in copies:
                cp.start()
            for cp in copies:
                cp.wait()
            out_cp(i, 0).start()
            out_cp(i, 0).wait()
            return carry

        lax.fori_loop(0, n_ch, body, 0)

    return k(table, idx_flat)


def _edge_preact(ag, a_blk, b3, ws, wn, wb, b, nb, m):
    two_f = ws.shape[1]
    bf = jnp.bfloat16
    s = jnp.dot(a_blk.astype(bf), ws, preferred_element_type=jnp.float32)
    ynb = jnp.dot(ag.astype(bf), wn, preferred_element_type=jnp.float32)
    q = jnp.dot(b3, wb, preferred_element_type=jnp.float32)
    s_exp = jnp.broadcast_to(s[:, None, :], (nb, m, two_f)).reshape(nb * m, two_f)
    return ynb + q + s_exp + b


def _stats_body(nb, m, ag_ref, a_ref, b3_ref, ws_ref, wn_ref, wb_ref,
                b_ref, out_ref):
    y = _edge_preact(ag_ref[...], a_ref[...], b3_ref[...],
                     ws_ref[...], wn_ref[...], wb_ref[...], b_ref[...], nb, m)

    @pl.when(pl.program_id(0) == 0)
    def _():
        out_ref[...] = jnp.zeros_like(out_ref)

    ssum = jnp.sum(y, axis=0, keepdims=True)
    ssq = jnp.sum(y * y, axis=0, keepdims=True)
    out_ref[...] += jnp.concatenate([ssum, ssq], axis=0)


def _main_body(nb, m, e_total, stats_ref, ag_ref, a_ref, b3_ref, ws_ref,
               wn_ref, wb_ref, b_ref, g1_ref, b1_ref, out_ref):
    y = _edge_preact(ag_ref[...], a_ref[...], b3_ref[...],
                     ws_ref[...], wn_ref[...], wb_ref[...], b_ref[...], nb, m)
    stats = stats_ref[...]
    mean = stats[0:1, :] / e_total
    var = stats[1:2, :] / e_total - mean * mean
    inv = lax.rsqrt(var + 1e-5)
    z = (y - mean) * (inv * g1_ref[...]) + b1_ref[...]
    f = z.shape[1] // 2
    filt = jax.nn.sigmoid(z[:, :f])
    core = jax.nn.softplus(z[:, f:])
    p = (filt * core).reshape(nb, m, f)
    out_ref[...] = jnp.sum(p, axis=1)


def _final_body(n, a_ref, ps_ref, g2_ref, b2_ref, out_ref):
    x = ps_ref[...]
    mean = jnp.mean(x, axis=0, keepdims=True)
    d = x - mean
    var = jnp.mean(d * d, axis=0, keepdims=True)
    z = d * lax.rsqrt(var + 1e-5) * g2_ref[...] + b2_ref[...]
    out_ref[...] = jax.nn.softplus(a_ref[...] + z)


def _tc_pipeline(ag, a, b3, ws, wn, wb, b, g1, b1, g2, b2,
                 interpret=False):
    n, f = a.shape
    e_total = ag.shape[0]
    m = e_total // n
    two_f = 2 * f
    de = b3.shape[1]
    nb = 200
    r = nb * m
    grid = n // nb

    full = lambda shape: pl.BlockSpec(shape, lambda i: (0,) * len(shape))
    stats = pl.pallas_call(
        functools.partial(_stats_body, nb, m),
        grid=(grid,),
        in_specs=[
            pl.BlockSpec((r, f), lambda i: (i, 0)),
            pl.BlockSpec((nb, f), lambda i: (i, 0)),
            pl.BlockSpec((r, de), lambda i: (i, 0)),
            full((f, two_f)), full((f, two_f)), full((de, two_f)),
            full((1, two_f)),
        ],
        out_specs=pl.BlockSpec((2, two_f), lambda i: (0, 0)),
        out_shape=jax.ShapeDtypeStruct((2, two_f), jnp.float32),
        interpret=interpret,
    )(ag, a, b3, ws, wn, wb, b)

    presum = pl.pallas_call(
        functools.partial(_main_body, nb, m, float(e_total)),
        grid=(grid,),
        in_specs=[
            full((2, two_f)),
            pl.BlockSpec((r, f), lambda i: (i, 0)),
            pl.BlockSpec((nb, f), lambda i: (i, 0)),
            pl.BlockSpec((r, de), lambda i: (i, 0)),
            full((f, two_f)), full((f, two_f)), full((de, two_f)),
            full((1, two_f)), full((1, two_f)), full((1, two_f)),
        ],
        out_specs=pl.BlockSpec((nb, f), lambda i: (i, 0)),
        out_shape=jax.ShapeDtypeStruct((n, f), jnp.float32),
        interpret=interpret,
    )(stats, ag, a, b3, ws, wn, wb, b, g1, b1)

    out = pl.pallas_call(
        functools.partial(_final_body, n),
        out_shape=jax.ShapeDtypeStruct((n, f), jnp.float32),
        interpret=interpret,
    )(a, presum, g2, b2)
    return out


def kernel(atom_features, bond_features, W, b, bn1_scale, bn1_bias,
           bn2_scale, bn2_bias, neighbor_indices):
    a = atom_features
    n, f = a.shape
    m = neighbor_indices.shape[1]
    idx_flat = neighbor_indices.reshape(-1)
    de = bond_features.shape[2]
    bflat = bond_features.reshape(n * m, de).astype(jnp.bfloat16)
    w16 = W.astype(jnp.bfloat16)
    ws, wn, wb = w16[:f], w16[f:2 * f], w16[2 * f:]

    ag = _sc_gather(a, idx_flat)
    return _tc_pipeline(
        ag, a, bflat, ws, wn, wb,
        b.reshape(1, -1), bn1_scale.reshape(1, -1), bn1_bias.reshape(1, -1),
        bn2_scale.reshape(1, -1), bn2_bias.reshape(1, -1))


# speedup vs baseline: 1.0227x; 1.0002x over previous
import functools

import jax
import jax.numpy as jnp
from jax import lax
from jax.experimental import pallas as pl
from jax.experimental.pallas import tpu as pltpu
from jax.experimental.pallas import tpu_sc as plsc

_NC, _NS = 2, 16
_NW = _NC * _NS
_CH = 400
_SUB = 400


def _sc_gather(table, idx_flat):
    e_total = idx_flat.shape[0]
    f = table.shape[1]
    dt = table.dtype
    rows_per_w = e_total // _NW
    n_ch = rows_per_w // _CH
    n_sub = _CH // _SUB
    mesh = plsc.VectorSubcoreMesh(core_axis_name="c", subcore_axis_name="s",
                                  num_cores=_NC, num_subcores=_NS)

    @functools.partial(
        pl.kernel, mesh=mesh,
        out_type=jax.ShapeDtypeStruct((e_total, f), dt),
        scratch_types=[
            pltpu.VMEM((_CH,), jnp.int32),
            pltpu.VMEM((_CH,), jnp.int32),
            pltpu.VMEM((_CH, f), dt),
            pltpu.VMEM((_CH, f), dt),
            pltpu.SemaphoreType.DMA,
            pltpu.SemaphoreType.DMA,
            pltpu.SemaphoreType.DMA,
        ],
        compiler_params=pltpu.CompilerParams(use_tc_tiling_on_sc=True),
    )
    def k(table_hbm, idx_hbm, out_hbm, idx_v0, idx_v1, rows_v0, rows_v1,
          isem, gsem, osem):
        wid = lax.axis_index("s") * _NC + lax.axis_index("c")
        base = wid * rows_per_w
        idx_vs = (idx_v0, idx_v1)
        rows_vs = (rows_v0, rows_v1)

        def idx_cp(i, slot):
            return pltpu.make_async_copy(
                idx_hbm.at[pl.ds(base + i * _CH, _CH)], idx_vs[slot], isem)

        def gather_cps(slot):
            return [pltpu.make_async_copy(
                table_hbm.at[idx_vs[slot].at[pl.ds(j * _SUB, _SUB)]],
                rows_vs[slot].at[pl.ds(j * _SUB, _SUB)], gsem)
                for j in range(n_sub)]

        def out_cp(i, slot):
            return pltpu.make_async_copy(
                rows_vs[slot], out_hbm.at[pl.ds(base + i * _CH, _CH)], osem)

        def body(i, carry):
            idx_cp(i, 0).start()
            idx_cp(i, 0).wait()
            copies = gather_cps(0)
            for cp in copies:
                cp.start()
            for cp in copies:
                cp.wait()
            out_cp(i, 0).start()
            out_cp(i, 0).wait()
            return carry

        lax.fori_loop(0, n_ch, body, 0)

    return k(table, idx_flat)


def _edge_preact(ag, a_blk, b3, ws, wn, wb, b, nb, m):
    two_f = ws.shape[1]
    bf = jnp.bfloat16
    s = jnp.dot(a_blk.astype(bf), ws, preferred_element_type=jnp.float32)
    ynb = jnp.dot(ag.astype(bf), wn, preferred_element_type=jnp.float32)
    q = jnp.dot(b3, wb, preferred_element_type=jnp.float32)
    s_exp = jnp.broadcast_to(s[:, None, :], (nb, m, two_f)).reshape(nb * m, two_f)
    return ynb + q + s_exp + b


def _stats_body(nb, m, ag_ref, a_ref, b3_ref, ws_ref, wn_ref, wb_ref,
                b_ref, out_ref):
    y = _edge_preact(ag_ref[...], a_ref[...], b3_ref[...],
                     ws_ref[...], wn_ref[...], wb_ref[...], b_ref[...], nb, m)

    @pl.when(pl.program_id(0) == 0)
    def _():
        out_ref[...] = jnp.zeros_like(out_ref)

    ssum = jnp.sum(y, axis=0, keepdims=True)
    ssq = jnp.sum(y * y, axis=0, keepdims=True)
    out_ref[...] += jnp.concatenate([ssum, ssq], axis=0)


def _main_body(nb, m, e_total, stats_ref, ag_ref, a_ref, b3_ref, ws_ref,
               wn_ref, wb_ref, b_ref, g1_ref, b1_ref, out_ref):
    y = _edge_preact(ag_ref[...], a_ref[...], b3_ref[...],
                     ws_ref[...], wn_ref[...], wb_ref[...], b_ref[...], nb, m)
    stats = stats_ref[...]
    mean = stats[0:1, :] / e_total
    var = stats[1:2, :] / e_total - mean * mean
    inv = lax.rsqrt(var + 1e-5)
    z = (y - mean) * (inv * g1_ref[...]) + b1_ref[...]
    f = z.shape[1] // 2
    filt = jax.nn.sigmoid(z[:, :f])
    core = jax.nn.softplus(z[:, f:])
    p = (filt * core).reshape(nb, m, f)
    out_ref[...] = jnp.sum(p, axis=1)


def _final_body(n, a_ref, ps_ref, g2_ref, b2_ref, out_ref):
    x = ps_ref[...]
    mean = jnp.mean(x, axis=0, keepdims=True)
    d = x - mean
    var = jnp.mean(d * d, axis=0, keepdims=True)
    z = d * lax.rsqrt(var + 1e-5) * g2_ref[...] + b2_ref[...]
    out_ref[...] = jax.nn.softplus(a_ref[...] + z)


def _tc_pipeline(ag, a, b3, ws, wn, wb, b, g1, b1, g2, b2,
                 interpret=False):
    n, f = a.shape
    e_total = ag.shape[0]
    m = e_total // n
    two_f = 2 * f
    de = b3.shape[1]
    nb = 200
    r = nb * m
    grid = n // nb

    full = lambda shape: pl.BlockSpec(shape, lambda i: (0,) * len(shape))
    stats = pl.pallas_call(
        functools.partial(_stats_body, nb, m),
        grid=(grid,),
        in_specs=[
            pl.BlockSpec((r, f), lambda i: (i, 0)),
            pl.BlockSpec((nb, f), lambda i: (i, 0)),
            pl.BlockSpec((r, de), lambda i: (i, 0)),
            full((f, two_f)), full((f, two_f)), full((de, two_f)),
            full((1, two_f)),
        ],
        out_specs=pl.BlockSpec((2, two_f), lambda i: (0, 0)),
        out_shape=jax.ShapeDtypeStruct((2, two_f), jnp.float32),
        interpret=interpret,
    )(ag, a, b3, ws, wn, wb, b)

    presum = pl.pallas_call(
        functools.partial(_main_body, nb, m, float(e_total)),
        grid=(grid,),
        in_specs=[
            full((2, two_f)),
            pl.BlockSpec((r, f), lambda i: (i, 0)),
            pl.BlockSpec((nb, f), lambda i: (i, 0)),
            pl.BlockSpec((r, de), lambda i: (i, 0)),
            full((f, two_f)), full((f, two_f)), full((de, two_f)),
            full((1, two_f)), full((1, two_f)), full((1, two_f)),
        ],
        out_specs=pl.BlockSpec((nb, f), lambda i: (i, 0)),
        out_shape=jax.ShapeDtypeStruct((n, f), jnp.float32),
        interpret=interpret,
    )(stats, ag, a, b3, ws, wn, wb, b, g1, b1)

    out = pl.pallas_call(
        functools.partial(_final_body, n),
        out_shape=jax.ShapeDtypeStruct((n, f), jnp.float32),
        interpret=interpret,
    )(a, presum, g2, b2)
    return out


def kernel(atom_features, bond_features, W, b, bn1_scale, bn1_bias,
           bn2_scale, bn2_bias, neighbor_indices):
    a = atom_features
    n, f = a.shape
    m = neighbor_indices.shape[1]
    idx_flat = neighbor_indices.reshape(-1)
    de = bond_features.shape[2]
    bflat = bond_features.reshape(n * m, de).astype(jnp.bfloat16)
    w16 = W.astype(jnp.bfloat16)
    ws, wn, wb = w16[:f], w16[f:2 * f], w16[2 * f:]

    ag = _sc_gather(a, idx_flat)
    return _tc_pipeline(
        ag, a, bflat, ws, wn, wb,
        b.reshape(1, -1), bn1_scale.reshape(1, -1), bn1_bias.reshape(1, -1),
        bn2_scale.reshape(1, -1), bn2_bias.reshape(1, -1))
